# speedup vs baseline: 1.0253x; 1.0253x over previous
import functools

import jax
import jax.numpy as jnp
from jax import lax
from jax.experimental import pallas as pl
from jax.experimental.pallas import tpu as pltpu
from jax.experimental.pallas import tpu_sc as plsc

_B = 4
_S = 2048
_D = 768
_N = _B * _S
_NC = 2
_NS = 16
_NW = _NC * _NS
_PER_W = _N // _NW
_C = 64
_NCHUNK = _PER_W // _C
_LANES = _D // 16


def _body(idx_hbm, table_hbm, pos_hbm, out_hbm, idx_v, rows_v, pos_v, sem):
    wid = lax.axis_index("s") * _NC + lax.axis_index("c")
    base = wid * _PER_W
    pos_base = base % _S

    def chunk(i, carry):
        rbase = base + i * _C
        pltpu.sync_copy(idx_hbm.at[pl.ds(rbase, _C)], idx_v)
        pltpu.async_copy(table_hbm.at[idx_v], rows_v, sem).wait()
        pltpu.sync_copy(pos_hbm.at[pl.ds(pos_base + i * _C, _C)], pos_v)

        def add_row(r, c):
            for j in range(_LANES):
                sl = pl.ds(j * 16, 16)
                rows_v[r, sl] = rows_v[r, sl] + pos_v[r, sl]
            return c

        lax.fori_loop(0, _C, add_row, 0)
        pltpu.sync_copy(rows_v, out_hbm.at[pl.ds(rbase, _C)])
        return carry

    lax.fori_loop(0, _NCHUNK, chunk, 0)


@jax.jit
def kernel(inputs, embed_table, pos_table):
    idx = inputs.reshape(_N).astype(jnp.int32)
    mesh = plsc.VectorSubcoreMesh(core_axis_name="c", subcore_axis_name="s")
    out = pl.kernel(
        _body,
        out_type=jax.ShapeDtypeStruct((_N, _D), jnp.float32),
        mesh=mesh,
        scratch_types=[
            pltpu.VMEM((_C,), jnp.int32),
            pltpu.VMEM((_C, _D), jnp.float32),
            pltpu.VMEM((_C, _D), jnp.float32),
            pltpu.SemaphoreType.DMA,
        ],
    )(idx, embed_table, pos_table)
    return out.reshape(_B, _S, _D)


# speedup vs baseline: 1.2722x; 1.2408x over previous
import jax
import jax.numpy as jnp
from jax import lax
from jax.experimental import pallas as pl
from jax.experimental.pallas import tpu as pltpu
from jax.experimental.pallas import tpu_sc as plsc

_B = 4
_S = 2048
_D = 768
_NC = 2
_NS = 16
_NW = _NC * _NS
_SW = _S // _NW
_C = 8
_NCHUNK = _SW // _C
_J = _D // 16


def _body(idx_hbm, table_hbm, pos_hbm, out_hbm, idx_v, in_v, pos_v, out_v,
          sem_g, sem_o):
    wid = lax.axis_index("s") * _NC + lax.axis_index("c")
    s_base = wid * _SW

    for b in range(_B):
        pltpu.sync_copy(idx_hbm.at[b, pl.ds(s_base, _SW)], idx_v.at[b])

    gathers = {}
    stores = {}

    def start(g):
        slot = g % 2
        cps = [
            pltpu.async_copy(
                table_hbm.at[idx_v.at[b, pl.ds(g * _C, _C)]],
                in_v.at[slot, b], sem_g)
            for b in range(_B)
        ]
        cps.append(pltpu.async_copy(
            pos_hbm.at[pl.ds(s_base + g * _C, _C)], pos_v.at[slot], sem_g))
        gathers[g] = cps

    start(0)
    start(1)
    for g in range(_NCHUNK):
        slot = g % 2
        for cp in gathers.pop(g):
            cp.wait()
        if g >= 2:
            for cp in stores.pop(g - 2):
                cp.wait()

        def add_s(s, c, slot=slot):
            for j in range(_J):
                sl = pl.ds(j * 16, 16)
                p = pos_v[slot, s, sl]
                for b in range(_B):
                    out_v[slot, b, s, sl] = in_v[slot, b, s, sl] + p
            return c

        lax.fori_loop(0, _C, add_s, 0)

        stores[g] = [
            pltpu.async_copy(
                out_v.at[slot, b],
                out_hbm.at[b, pl.ds(s_base + g * _C, _C)], sem_o)
            for b in range(_B)
        ]
        if g + 2 < _NCHUNK:
            start(g + 2)
    for g in (_NCHUNK - 2, _NCHUNK - 1):
        for cp in stores.pop(g):
            cp.wait()


@jax.jit
def kernel(inputs, embed_table, pos_table):
    idx = inputs.astype(jnp.int32)
    mesh = plsc.VectorSubcoreMesh(core_axis_name="c", subcore_axis_name="s")
    out = pl.kernel(
        _body,
        out_type=jax.ShapeDtypeStruct((_B, _S, _D), jnp.float32),
        mesh=mesh,
        scratch_types=[
            pltpu.VMEM((_B, _SW), jnp.int32),
            pltpu.VMEM((2, _B, _C, _D), jnp.float32),
            pltpu.VMEM((2, _C, _D), jnp.float32),
            pltpu.VMEM((2, _B, _C, _D), jnp.float32),
            pltpu.SemaphoreType.DMA,
            pltpu.SemaphoreType.DMA,
        ],
    )(idx, embed_table, pos_table)
    return out


# speedup vs baseline: 1.5382x; 1.2091x over previous
import jax
import jax.numpy as jnp
from jax import lax
from jax.experimental import pallas as pl
from jax.experimental.pallas import tpu as pltpu
from jax.experimental.pallas import tpu_sc as plsc

_B = 4
_S = 2048
_D = 768
_NC = 2
_NS = 16
_NW = _NC * _NS
_SW = _S // _NW
_C = 8
_R = _C * _B
_NCHUNK = _SW // _C
_J = _D // 16


def _body(idx_hbm, table_hbm, pos_hbm, out_hbm, idx_s, in_v, pos_v,
          out_v, sem_g, sem_o):
    wid = lax.axis_index("s") * _NC + lax.axis_index("c")
    s_base = wid * _SW

    pltpu.sync_copy(idx_hbm.at[pl.ds(s_base * _B, _SW * _B)], idx_s)

    gathers = {}
    stores = {}

    def start(g):
        slot = g % 2
        gathers[g] = [
            pltpu.async_copy(
                table_hbm.at[idx_s.at[pl.ds(g * _R, _R)]], in_v.at[slot],
                sem_g),
            pltpu.async_copy(
                pos_hbm.at[pl.ds(s_base + g * _C, _C)], pos_v.at[slot],
                sem_g),
        ]

    start(0)
    start(1)
    for g in range(_NCHUNK):
        slot = g % 2
        for cp in gathers.pop(g):
            cp.wait()
        if g >= 2:
            for cp in stores.pop(g - 2):
                cp.wait()

        def add_s(s, c, slot=slot):
            @plsc.parallel_loop(0, _J, 1, unroll=8)
            def add_j(j):
                sl = pl.ds(j * 16, 16)
                p = pos_v[slot, s, sl]
                for b in range(_B):
                    out_v[slot, b, s, sl] = in_v[slot, s * _B + b, sl] + p
            return c

        lax.fori_loop(0, _C, add_s, 0)

        stores[g] = [
            pltpu.async_copy(
                out_v.at[slot, b],
                out_hbm.at[b, pl.ds(s_base + g * _C, _C)], sem_o)
            for b in range(_B)
        ]
        if g + 2 < _NCHUNK:
            start(g + 2)
    for g in (_NCHUNK - 2, _NCHUNK - 1):
        for cp in stores.pop(g):
            cp.wait()


@jax.jit
def kernel(inputs, embed_table, pos_table):
    idx = jnp.transpose(inputs).reshape(_S * _B).astype(jnp.int32)
    mesh = plsc.VectorSubcoreMesh(core_axis_name="c", subcore_axis_name="s")
    out = pl.kernel(
        _body,
        out_type=jax.ShapeDtypeStruct((_B, _S, _D), jnp.float32),
        mesh=mesh,
        scratch_types=[
            pltpu.VMEM((_SW * _B,), jnp.int32),
            pltpu.VMEM((2, _R, _D), jnp.float32),
            pltpu.VMEM((2, _C, _D), jnp.float32),
            pltpu.VMEM((2, _B, _C, _D), jnp.float32),
            pltpu.SemaphoreType.DMA,
            pltpu.SemaphoreType.DMA,
        ],
    )(idx, embed_table, pos_table)
    return out
